# baseline (speedup 1.0000x reference)
import dataclasses
import functools

import jax
import jax.numpy as jnp
from jax import lax
from jax.experimental import pallas as pl
from jax.experimental.pallas import tpu as pltpu
from jax.experimental.pallas import tpu_sc as plsc

B = 64
N = 32768
C = 16
K = 4

BSC = 24
BTC = B - BSC

NW = 32
NSS = BSC * C
SPW = NSS // NW
NT = 256
ROWS = NT
BLKV = 32
RPB = BLKV // 8
NBLK = (NT * 128) // (BLKV * 16)
NEG = float("-inf")
POS = float("inf")


def _tree_reduce(vals, op):
    while len(vals) > 1:
        nxt = [op(vals[i], vals[i + 1]) for i in range(0, len(vals) - 1, 2)]
        if len(vals) % 2:
            nxt.append(vals[-1])
        vals = nxt
    return vals[0]


def _insert_top(t, v):
    out = []
    cur = v
    for i in range(3):
        out.append(jnp.maximum(t[i], cur))
        cur = jnp.minimum(t[i], cur)
    out.append(jnp.maximum(t[3], cur))
    return out


def _insert_bot(bt, v):
    out = []
    cur = v
    for i in range(3):
        out.append(jnp.minimum(bt[i], cur))
        cur = jnp.maximum(bt[i], cur)
    out.append(jnp.minimum(bt[3], cur))
    return out


def _ladder_ids(vals, ids, cur, cid, gt):
    o_v, o_i = [], []
    for r in range(3):
        c = gt(cur, vals[r])
        o_v.append(jnp.where(c, cur, vals[r]))
        o_i.append(jnp.where(c, cid, ids[r]))
        cur, cid = jnp.where(c, vals[r], cur), jnp.where(c, ids[r], cid)
    c = gt(cur, vals[3])
    o_v.append(jnp.where(c, cur, vals[3]))
    o_i.append(jnp.where(c, cid, ids[3]))
    return o_v, o_i


def _sortd(v):
    return plsc.sort_key_val(v, v, descending=True)[0]


def _sorta(v):
    return plsc.sort_key_val(v, v, descending=False)[0]


def _milpool_sc(x5):
    mesh = plsc.VectorSubcoreMesh(core_axis_name="c", subcore_axis_name="s")
    cp = pltpu.CompilerParams()
    for fld, val in (("needs_layout_passes", False),
                     ("use_tc_tiling_on_sc", False)):
        if fld in pltpu.CompilerParams.__dataclass_fields__:
            cp = dataclasses.replace(cp, **{fld: val})

    @functools.partial(
        pl.kernel,
        compiler_params=cp,
        out_type=jax.ShapeDtypeStruct((NSS * 2, 16), jnp.float32),
        mesh=mesh,
        scratch_types=[
            pltpu.VMEM((ROWS, 128), jnp.float32),
            pltpu.VMEM((ROWS, 128), jnp.float32),
            pltpu.VMEM((SPW * 2, 16), jnp.float32),
            pltpu.SemaphoreType.DMA,
            pltpu.SemaphoreType.DMA,
        ],
    )
    def k(x_hbm, o_hbm, buf_a, buf_b, ost, sem_a, sem_b):
        wid = lax.axis_index("s") * 2 + lax.axis_index("c")
        ss0 = wid * SPW
        iota = lax.iota(jnp.int32, 16)
        lt4 = iota < 4
        cols = [iota + 16 * q for q in range(8)]
        ninf = jnp.full((16,), NEG, jnp.float32)
        pinf = jnp.full((16,), POS, jnp.float32)
        zid = jnp.full((16,), 0, jnp.int32)

        def src(ss):
            b = ss // 16
            ct = (ss // 8) % 2
            cr = ss % 8
            return x_hbm.at[b, ct, pl.ds(0, NT), cr, :]

        def process(buf, si):
            def half(blk, tv, ti, bv, bi):
                r0 = blk * RPB
                vs = [buf[r0 + (q // 8), pl.ds(16 * (q % 8), 16)]
                      for q in range(BLKV)]
                bm = _tree_reduce(list(vs), jnp.maximum)
                bn = _tree_reduce(list(vs), jnp.minimum)
                cid = zid + blk
                tv, ti = _ladder_ids(tv, ti, bm, cid, lambda a, b: a > b)
                bv, bi = _ladder_ids(bv, bi, bn, cid, lambda a, b: a < b)
                return tv, ti, bv, bi

            def body(j, carry):
                tv = list(carry[0:4])
                ti = list(carry[4:8])
                bv = list(carry[8:12])
                bi = list(carry[12:16])
                tv, ti, bv, bi = half(j * 2, tv, ti, bv, bi)
                tv, ti, bv, bi = half(j * 2 + 1, tv, ti, bv, bi)
                return tuple(tv) + tuple(ti) + tuple(bv) + tuple(bi)

            carry = lax.fori_loop(
                0, NBLK // 2, body,
                (ninf, ninf, ninf, ninf, zid, zid, zid, zid,
                 pinf, pinf, pinf, pinf, zid, zid, zid, zid))
            top_ids = carry[4:8]
            bot_ids = carry[12:16]

            t = [ninf, ninf, ninf, ninf]
            for r in range(4):
                row0 = top_ids[r] * RPB
                for q in range(BLKV):
                    rowv = row0 + (q // 8)
                    v = plsc.load_gather(buf, [rowv, cols[q % 8]])
                    t = _insert_top(t, v)
            bt = [pinf, pinf, pinf, pinf]
            for r in range(4):
                row0 = bot_ids[r] * RPB
                for q in range(BLKV):
                    rowv = row0 + (q // 8)
                    v = plsc.load_gather(buf, [rowv, cols[q % 8]])
                    bt = _insert_bot(bt, v)

            s = _sortd(t[0])
            for v in t[1:]:
                s = _sortd(jnp.where(lt4, s, jnp.flip(_sortd(v))))
            sb = _sorta(bt[0])
            for v in bt[1:]:
                sb = _sorta(jnp.where(lt4, sb, jnp.flip(_sorta(v))))

            ost[2 * si, :] = s
            ost[2 * si + 1, :] = sb

        pltpu.async_copy(src(ss0), buf_a, sem_a)
        pltpu.async_copy(src(ss0 + 1), buf_b, sem_b)

        @pl.loop(0, SPW // 2)
        def _pair(p):
            ssa = ss0 + 2 * p
            pltpu.make_async_copy(src(ssa), buf_a, sem_a).wait()
            process(buf_a, 2 * p)

            @pl.when(2 * p + 2 < SPW)
            def _refill_a():
                pltpu.async_copy(src(ssa + 2), buf_a, sem_a)

            pltpu.make_async_copy(src(ssa + 1), buf_b, sem_b).wait()
            process(buf_b, 2 * p + 1)

            @pl.when(2 * p + 3 < SPW)
            def _refill_b():
                pltpu.async_copy(src(ssa + 3), buf_b, sem_b)

        pltpu.sync_copy(ost, o_hbm.at[pl.ds(ss0 * 2, SPW * 2), :])

    return k(x5)


def _c2d(a, b):
    return jnp.maximum(a, b), jnp.minimum(a, b)


def _c2a(a, b):
    return jnp.minimum(a, b), jnp.maximum(a, b)


def _sort4(vs, c2):
    a, b, c, d = vs
    a, b = c2(a, b)
    c, d = c2(c, d)
    a, c = c2(a, c)
    b, d = c2(b, d)
    b, c = c2(b, c)
    return [a, b, c, d]


def _merge4(x, y, ext, c2):
    z1 = ext(x[0], y[3])
    z2 = ext(x[1], y[2])
    z3 = ext(x[2], y[1])
    z4 = ext(x[3], y[0])
    z1, z3 = c2(z1, z3)
    z2, z4 = c2(z2, z4)
    z1, z2 = c2(z1, z2)
    z3, z4 = c2(z3, z4)
    return [z1, z2, z3, z4]


def _tc_body(x_ref, o_ref):
    iota = lax.broadcasted_iota(jnp.int32, (8, 128), 1)
    ninf = jnp.full((8, 128), NEG, jnp.float32)
    pinf = jnp.full((8, 128), POS, jnp.float32)

    def pass_both(ct):
        top, bot = None, None
        for g in range(8):
            leaves = []
            for j in range(8):
                r0 = g * 32 + j * 4
                vs = [x_ref[0, ct, r0 + i] for i in range(4)]
                leaves.append(_sort4(vs, _c2d))
            tsubs = list(leaves)
            while len(tsubs) > 1:
                tsubs = [_merge4(tsubs[i], tsubs[i + 1], jnp.maximum, _c2d)
                         for i in range(0, len(tsubs), 2)]
            bsubs = [lv[::-1] for lv in leaves]
            while len(bsubs) > 1:
                bsubs = [_merge4(bsubs[i], bsubs[i + 1], jnp.minimum, _c2a)
                         for i in range(0, len(bsubs), 2)]
            top = (tsubs[0] if top is None
                   else _merge4(top, tsubs[0], jnp.maximum, _c2d))
            bot = (bsubs[0] if bot is None
                   else _merge4(bot, bsubs[0], jnp.minimum, _c2a))
        return top, bot

    for ct in range(2):
        t, bt = pass_both(ct)
        for k in range(4):
            m = jnp.max(t[0], axis=1, keepdims=True)
            mb = jnp.broadcast_to(m, (8, 128))
            li = jnp.min(jnp.where(t[0] == mb, iota, 128), axis=1,
                         keepdims=True)
            sel = iota == jnp.broadcast_to(li, (8, 128))
            o_ref[0, ct, 0, k] = mb
            t = [jnp.where(sel, t[1], t[0]), jnp.where(sel, t[2], t[1]),
                 jnp.where(sel, t[3], t[2]), jnp.where(sel, ninf, t[3])]
        for k in range(4):
            m = jnp.min(bt[0], axis=1, keepdims=True)
            mb = jnp.broadcast_to(m, (8, 128))
            li = jnp.min(jnp.where(bt[0] == mb, iota, 128), axis=1,
                         keepdims=True)
            sel = iota == jnp.broadcast_to(li, (8, 128))
            o_ref[0, ct, 1, k] = mb
            bt = [jnp.where(sel, bt[1], bt[0]), jnp.where(sel, bt[2], bt[1]),
                  jnp.where(sel, bt[3], bt[2]), jnp.where(sel, pinf, bt[3])]


def _milpool_tc(x5):
    return pl.pallas_call(
        _tc_body,
        grid=(BTC,),
        in_specs=[pl.BlockSpec((1, 2, NT, 8, 128),
                               lambda i: (i + BSC, 0, 0, 0, 0))],
        out_specs=pl.BlockSpec((1, 2, 2, 4, 8, 128),
                               lambda i: (i, 0, 0, 0, 0, 0)),
        out_shape=jax.ShapeDtypeStruct((BTC, 2, 2, 4, 8, 128), jnp.float32),
    )(x5)


@jax.jit
def kernel(inputs):
    x5 = inputs.reshape(B, NT, 128, 2, 8).transpose(0, 3, 1, 4, 2)
    o = _milpool_sc(x5)
    otc = _milpool_tc(x5)
    o6 = o.reshape(BSC, 2, 8, 2, 16)[:, :, :, :, :K]
    out_sc = jnp.transpose(o6, (0, 3, 4, 1, 2)).reshape(BSC, 2 * K, C)
    o7 = otc[:, :, :, :, :, 0]
    out_tc = jnp.transpose(o7, (0, 2, 3, 1, 4)).reshape(BTC, 2 * K, C)
    return jnp.concatenate([out_sc, out_tc], axis=0)


# speedup vs baseline: 1.1028x; 1.1028x over previous
import dataclasses
import functools

import jax
import jax.numpy as jnp
from jax import lax
from jax.experimental import pallas as pl
from jax.experimental.pallas import tpu as pltpu
from jax.experimental.pallas import tpu_sc as plsc

B = 64
N = 32768
C = 16
K = 4

BSC = 40
BTC = B - BSC

NW = 32
NSS = BSC * C
SPW = NSS // NW
NT = 256
ROWS = NT
BLKV = 32
RPB = BLKV // 8
NBLK = (NT * 128) // (BLKV * 16)
NEG = float("-inf")
POS = float("inf")


def _tree_reduce(vals, op):
    while len(vals) > 1:
        nxt = [op(vals[i], vals[i + 1]) for i in range(0, len(vals) - 1, 2)]
        if len(vals) % 2:
            nxt.append(vals[-1])
        vals = nxt
    return vals[0]


def _insert_top(t, v):
    out = []
    cur = v
    for i in range(3):
        out.append(jnp.maximum(t[i], cur))
        cur = jnp.minimum(t[i], cur)
    out.append(jnp.maximum(t[3], cur))
    return out


def _insert_bot(bt, v):
    out = []
    cur = v
    for i in range(3):
        out.append(jnp.minimum(bt[i], cur))
        cur = jnp.maximum(bt[i], cur)
    out.append(jnp.minimum(bt[3], cur))
    return out


def _ladder_ids(vals, ids, cur, cid, gt):
    o_v, o_i = [], []
    for r in range(3):
        c = gt(cur, vals[r])
        o_v.append(jnp.where(c, cur, vals[r]))
        o_i.append(jnp.where(c, cid, ids[r]))
        cur, cid = jnp.where(c, vals[r], cur), jnp.where(c, ids[r], cid)
    c = gt(cur, vals[3])
    o_v.append(jnp.where(c, cur, vals[3]))
    o_i.append(jnp.where(c, cid, ids[3]))
    return o_v, o_i


def _sortd(v):
    return plsc.sort_key_val(v, v, descending=True)[0]


def _sorta(v):
    return plsc.sort_key_val(v, v, descending=False)[0]


def _milpool_sc(x5):
    mesh = plsc.VectorSubcoreMesh(core_axis_name="c", subcore_axis_name="s")
    cp = pltpu.CompilerParams()
    for fld, val in (("needs_layout_passes", False),
                     ("use_tc_tiling_on_sc", False)):
        if fld in pltpu.CompilerParams.__dataclass_fields__:
            cp = dataclasses.replace(cp, **{fld: val})

    @functools.partial(
        pl.kernel,
        compiler_params=cp,
        out_type=jax.ShapeDtypeStruct((NSS * 2, 16), jnp.float32),
        mesh=mesh,
        scratch_types=[
            pltpu.VMEM((ROWS, 128), jnp.float32),
            pltpu.VMEM((ROWS, 128), jnp.float32),
            pltpu.VMEM((SPW * 2, 16), jnp.float32),
            pltpu.SemaphoreType.DMA,
            pltpu.SemaphoreType.DMA,
        ],
    )
    def k(x_hbm, o_hbm, buf_a, buf_b, ost, sem_a, sem_b):
        wid = lax.axis_index("s") * 2 + lax.axis_index("c")
        ss0 = wid * SPW
        iota = lax.iota(jnp.int32, 16)
        lt4 = iota < 4
        cols = [iota + 16 * q for q in range(8)]
        ninf = jnp.full((16,), NEG, jnp.float32)
        pinf = jnp.full((16,), POS, jnp.float32)
        zid = jnp.full((16,), 0, jnp.int32)

        def src(ss):
            b = ss // 16
            ct = (ss // 8) % 2
            cr = ss % 8
            return x_hbm.at[b, ct, pl.ds(0, NT), cr, :]

        def process(buf, si):
            def half(blk, tv, ti, bv, bi):
                r0 = blk * RPB
                vs = [buf[r0 + (q // 8), pl.ds(16 * (q % 8), 16)]
                      for q in range(BLKV)]
                bm = _tree_reduce(list(vs), jnp.maximum)
                bn = _tree_reduce(list(vs), jnp.minimum)
                cid = zid + blk
                tv, ti = _ladder_ids(tv, ti, bm, cid, lambda a, b: a > b)
                bv, bi = _ladder_ids(bv, bi, bn, cid, lambda a, b: a < b)
                return tv, ti, bv, bi

            def body(j, carry):
                tv = list(carry[0:4])
                ti = list(carry[4:8])
                bv = list(carry[8:12])
                bi = list(carry[12:16])
                tv, ti, bv, bi = half(j * 2, tv, ti, bv, bi)
                tv, ti, bv, bi = half(j * 2 + 1, tv, ti, bv, bi)
                return tuple(tv) + tuple(ti) + tuple(bv) + tuple(bi)

            carry = lax.fori_loop(
                0, NBLK // 2, body,
                (ninf, ninf, ninf, ninf, zid, zid, zid, zid,
                 pinf, pinf, pinf, pinf, zid, zid, zid, zid))
            top_ids = carry[4:8]
            bot_ids = carry[12:16]

            t = [ninf, ninf, ninf, ninf]
            for r in range(4):
                row0 = top_ids[r] * RPB
                for q in range(BLKV):
                    rowv = row0 + (q // 8)
                    v = plsc.load_gather(buf, [rowv, cols[q % 8]])
                    t = _insert_top(t, v)
            bt = [pinf, pinf, pinf, pinf]
            for r in range(4):
                row0 = bot_ids[r] * RPB
                for q in range(BLKV):
                    rowv = row0 + (q // 8)
                    v = plsc.load_gather(buf, [rowv, cols[q % 8]])
                    bt = _insert_bot(bt, v)

            s = _sortd(t[0])
            for v in t[1:]:
                s = _sortd(jnp.where(lt4, s, jnp.flip(_sortd(v))))
            sb = _sorta(bt[0])
            for v in bt[1:]:
                sb = _sorta(jnp.where(lt4, sb, jnp.flip(_sorta(v))))

            ost[2 * si, :] = s
            ost[2 * si + 1, :] = sb

        pltpu.async_copy(src(ss0), buf_a, sem_a)
        pltpu.async_copy(src(ss0 + 1), buf_b, sem_b)

        @pl.loop(0, SPW // 2)
        def _pair(p):
            ssa = ss0 + 2 * p
            pltpu.make_async_copy(src(ssa), buf_a, sem_a).wait()
            process(buf_a, 2 * p)

            @pl.when(2 * p + 2 < SPW)
            def _refill_a():
                pltpu.async_copy(src(ssa + 2), buf_a, sem_a)

            pltpu.make_async_copy(src(ssa + 1), buf_b, sem_b).wait()
            process(buf_b, 2 * p + 1)

            @pl.when(2 * p + 3 < SPW)
            def _refill_b():
                pltpu.async_copy(src(ssa + 3), buf_b, sem_b)

        pltpu.sync_copy(ost, o_hbm.at[pl.ds(ss0 * 2, SPW * 2), :])

    return k(x5)


def _c2d(a, b):
    return jnp.maximum(a, b), jnp.minimum(a, b)


def _c2a(a, b):
    return jnp.minimum(a, b), jnp.maximum(a, b)


def _sort4(vs, c2):
    a, b, c, d = vs
    a, b = c2(a, b)
    c, d = c2(c, d)
    a, c = c2(a, c)
    b, d = c2(b, d)
    b, c = c2(b, c)
    return [a, b, c, d]


def _merge4(x, y, ext, c2):
    z1 = ext(x[0], y[3])
    z2 = ext(x[1], y[2])
    z3 = ext(x[2], y[1])
    z4 = ext(x[3], y[0])
    z1, z3 = c2(z1, z3)
    z2, z4 = c2(z2, z4)
    z1, z2 = c2(z1, z2)
    z3, z4 = c2(z3, z4)
    return [z1, z2, z3, z4]


def _tc_body(x_ref, o_ref):
    iota = lax.broadcasted_iota(jnp.int32, (8, 128), 1)
    ninf = jnp.full((8, 128), NEG, jnp.float32)
    pinf = jnp.full((8, 128), POS, jnp.float32)

    def group(ct, g):
        leaves = []
        for j in range(8):
            r0 = g * 32 + j * 4
            vs = [x_ref[0, ct, r0 + i] for i in range(4)]
            leaves.append(_sort4(vs, _c2d))
        tsubs = list(leaves)
        while len(tsubs) > 1:
            tsubs = [_merge4(tsubs[i], tsubs[i + 1], jnp.maximum, _c2d)
                     for i in range(0, len(tsubs), 2)]
        bsubs = [lv[::-1] for lv in leaves]
        while len(bsubs) > 1:
            bsubs = [_merge4(bsubs[i], bsubs[i + 1], jnp.minimum, _c2a)
                     for i in range(0, len(bsubs), 2)]
        return tsubs[0], bsubs[0]

    top = [None, None]
    bot = [None, None]
    for g in range(8):
        for ct in range(2):
            tg, bg = group(ct, g)
            top[ct] = (tg if top[ct] is None
                       else _merge4(top[ct], tg, jnp.maximum, _c2d))
            bot[ct] = (bg if bot[ct] is None
                       else _merge4(bot[ct], bg, jnp.minimum, _c2a))

    for ct in range(2):
        t, bt = top[ct], bot[ct]
        for k in range(4):
            m = jnp.max(t[0], axis=1, keepdims=True)
            mb = jnp.broadcast_to(m, (8, 128))
            li = jnp.min(jnp.where(t[0] == mb, iota, 128), axis=1,
                         keepdims=True)
            sel = iota == jnp.broadcast_to(li, (8, 128))
            o_ref[0, ct, 0, k] = mb
            t = [jnp.where(sel, t[1], t[0]), jnp.where(sel, t[2], t[1]),
                 jnp.where(sel, t[3], t[2]), jnp.where(sel, ninf, t[3])]
        for k in range(4):
            m = jnp.min(bt[0], axis=1, keepdims=True)
            mb = jnp.broadcast_to(m, (8, 128))
            li = jnp.min(jnp.where(bt[0] == mb, iota, 128), axis=1,
                         keepdims=True)
            sel = iota == jnp.broadcast_to(li, (8, 128))
            o_ref[0, ct, 1, k] = mb
            bt = [jnp.where(sel, bt[1], bt[0]), jnp.where(sel, bt[2], bt[1]),
                  jnp.where(sel, bt[3], bt[2]), jnp.where(sel, pinf, bt[3])]


def _milpool_tc(x5):
    return pl.pallas_call(
        _tc_body,
        grid=(BTC,),
        in_specs=[pl.BlockSpec((1, 2, NT, 8, 128),
                               lambda i: (i + BSC, 0, 0, 0, 0))],
        out_specs=pl.BlockSpec((1, 2, 2, 4, 8, 128),
                               lambda i: (i, 0, 0, 0, 0, 0)),
        out_shape=jax.ShapeDtypeStruct((BTC, 2, 2, 4, 8, 128), jnp.float32),
    )(x5)


@jax.jit
def kernel(inputs):
    x5 = inputs.reshape(B, NT, 128, 2, 8).transpose(0, 3, 1, 4, 2)
    o = _milpool_sc(x5)
    otc = _milpool_tc(x5)
    o6 = o.reshape(BSC, 2, 8, 2, 16)[:, :, :, :, :K]
    out_sc = jnp.transpose(o6, (0, 3, 4, 1, 2)).reshape(BSC, 2 * K, C)
    o7 = otc[:, :, :, :, :, 0]
    out_tc = jnp.transpose(o7, (0, 2, 3, 1, 4)).reshape(BTC, 2 * K, C)
    return jnp.concatenate([out_sc, out_tc], axis=0)


# speedup vs baseline: 1.1401x; 1.0339x over previous
import dataclasses
import functools

import jax
import jax.numpy as jnp
from jax import lax
from jax.experimental import pallas as pl
from jax.experimental.pallas import tpu as pltpu
from jax.experimental.pallas import tpu_sc as plsc

B = 64
N = 32768
C = 16
K = 4

BSC = 32
BTC = B - BSC

NW = 32
NSS = BSC * C
SPW = NSS // NW
NT = 256
ROWS = NT
BLKV = 32
RPB = BLKV // 8
NBLK = (NT * 128) // (BLKV * 16)
NEG = float("-inf")
POS = float("inf")


def _tree_reduce(vals, op):
    while len(vals) > 1:
        nxt = [op(vals[i], vals[i + 1]) for i in range(0, len(vals) - 1, 2)]
        if len(vals) % 2:
            nxt.append(vals[-1])
        vals = nxt
    return vals[0]


def _insert_top(t, v):
    out = []
    cur = v
    for i in range(3):
        out.append(jnp.maximum(t[i], cur))
        cur = jnp.minimum(t[i], cur)
    out.append(jnp.maximum(t[3], cur))
    return out


def _insert_bot(bt, v):
    out = []
    cur = v
    for i in range(3):
        out.append(jnp.minimum(bt[i], cur))
        cur = jnp.maximum(bt[i], cur)
    out.append(jnp.minimum(bt[3], cur))
    return out


def _ladder_ids(vals, ids, cur, cid, gt):
    o_v, o_i = [], []
    for r in range(3):
        c = gt(cur, vals[r])
        o_v.append(jnp.where(c, cur, vals[r]))
        o_i.append(jnp.where(c, cid, ids[r]))
        cur, cid = jnp.where(c, vals[r], cur), jnp.where(c, ids[r], cid)
    c = gt(cur, vals[3])
    o_v.append(jnp.where(c, cur, vals[3]))
    o_i.append(jnp.where(c, cid, ids[3]))
    return o_v, o_i


def _sortd(v):
    return plsc.sort_key_val(v, v, descending=True)[0]


def _sorta(v):
    return plsc.sort_key_val(v, v, descending=False)[0]


def _milpool_sc(x5):
    mesh = plsc.VectorSubcoreMesh(core_axis_name="c", subcore_axis_name="s")
    cp = pltpu.CompilerParams()
    for fld, val in (("needs_layout_passes", False),
                     ("use_tc_tiling_on_sc", False)):
        if fld in pltpu.CompilerParams.__dataclass_fields__:
            cp = dataclasses.replace(cp, **{fld: val})

    @functools.partial(
        pl.kernel,
        compiler_params=cp,
        out_type=jax.ShapeDtypeStruct((NSS * 2, 16), jnp.float32),
        mesh=mesh,
        scratch_types=[
            pltpu.VMEM((ROWS, 128), jnp.float32),
            pltpu.VMEM((ROWS, 128), jnp.float32),
            pltpu.VMEM((SPW * 2, 16), jnp.float32),
            pltpu.SemaphoreType.DMA,
            pltpu.SemaphoreType.DMA,
        ],
    )
    def k(x_hbm, o_hbm, buf_a, buf_b, ost, sem_a, sem_b):
        wid = lax.axis_index("s") * 2 + lax.axis_index("c")
        ss0 = wid * SPW
        iota = lax.iota(jnp.int32, 16)
        lt4 = iota < 4
        cols = [iota + 16 * q for q in range(8)]
        ninf = jnp.full((16,), NEG, jnp.float32)
        pinf = jnp.full((16,), POS, jnp.float32)
        zid = jnp.full((16,), 0, jnp.int32)

        def src(ss):
            b = ss // 16
            ct = (ss // 8) % 2
            cr = ss % 8
            return x_hbm.at[b, ct, pl.ds(0, NT), cr, :]

        def process(buf, si):
            def half(blk, tv, ti, bv, bi):
                r0 = blk * RPB
                vs = [buf[r0 + (q // 8), pl.ds(16 * (q % 8), 16)]
                      for q in range(BLKV)]
                bm = _tree_reduce(list(vs), jnp.maximum)
                bn = _tree_reduce(list(vs), jnp.minimum)
                cid = zid + blk
                tv, ti = _ladder_ids(tv, ti, bm, cid, lambda a, b: a > b)
                bv, bi = _ladder_ids(bv, bi, bn, cid, lambda a, b: a < b)
                return tv, ti, bv, bi

            def body(j, carry):
                tv = list(carry[0:4])
                ti = list(carry[4:8])
                bv = list(carry[8:12])
                bi = list(carry[12:16])
                tv, ti, bv, bi = half(j * 2, tv, ti, bv, bi)
                tv, ti, bv, bi = half(j * 2 + 1, tv, ti, bv, bi)
                return tuple(tv) + tuple(ti) + tuple(bv) + tuple(bi)

            carry = lax.fori_loop(
                0, NBLK // 2, body,
                (ninf, ninf, ninf, ninf, zid, zid, zid, zid,
                 pinf, pinf, pinf, pinf, zid, zid, zid, zid))
            top_ids = carry[4:8]
            bot_ids = carry[12:16]

            t = [ninf, ninf, ninf, ninf]
            for r in range(4):
                row0 = top_ids[r] * RPB
                for q in range(BLKV):
                    rowv = row0 + (q // 8)
                    v = plsc.load_gather(buf, [rowv, cols[q % 8]])
                    t = _insert_top(t, v)
            bt = [pinf, pinf, pinf, pinf]
            for r in range(4):
                row0 = bot_ids[r] * RPB
                for q in range(BLKV):
                    rowv = row0 + (q // 8)
                    v = plsc.load_gather(buf, [rowv, cols[q % 8]])
                    bt = _insert_bot(bt, v)

            s = _sortd(t[0])
            for v in t[1:]:
                s = _sortd(jnp.where(lt4, s, jnp.flip(_sortd(v))))
            sb = _sorta(bt[0])
            for v in bt[1:]:
                sb = _sorta(jnp.where(lt4, sb, jnp.flip(_sorta(v))))

            ost[2 * si, :] = s
            ost[2 * si + 1, :] = sb

        pltpu.async_copy(src(ss0), buf_a, sem_a)
        pltpu.async_copy(src(ss0 + 1), buf_b, sem_b)

        @pl.loop(0, SPW // 2)
        def _pair(p):
            ssa = ss0 + 2 * p
            pltpu.make_async_copy(src(ssa), buf_a, sem_a).wait()
            process(buf_a, 2 * p)

            @pl.when(2 * p + 2 < SPW)
            def _refill_a():
                pltpu.async_copy(src(ssa + 2), buf_a, sem_a)

            pltpu.make_async_copy(src(ssa + 1), buf_b, sem_b).wait()
            process(buf_b, 2 * p + 1)

            @pl.when(2 * p + 3 < SPW)
            def _refill_b():
                pltpu.async_copy(src(ssa + 3), buf_b, sem_b)

        pltpu.sync_copy(ost, o_hbm.at[pl.ds(ss0 * 2, SPW * 2), :])

    return k(x5)


def _c2d(a, b):
    return jnp.maximum(a, b), jnp.minimum(a, b)


def _c2a(a, b):
    return jnp.minimum(a, b), jnp.maximum(a, b)


def _sort4(vs, c2):
    a, b, c, d = vs
    a, b = c2(a, b)
    c, d = c2(c, d)
    a, c = c2(a, c)
    b, d = c2(b, d)
    b, c = c2(b, c)
    return [a, b, c, d]


def _merge4(x, y, ext, c2):
    z1 = ext(x[0], y[3])
    z2 = ext(x[1], y[2])
    z3 = ext(x[2], y[1])
    z4 = ext(x[3], y[0])
    z1, z3 = c2(z1, z3)
    z2, z4 = c2(z2, z4)
    z1, z2 = c2(z1, z2)
    z3, z4 = c2(z3, z4)
    return [z1, z2, z3, z4]


def _tc_body(x_ref, o_ref):
    iota = lax.broadcasted_iota(jnp.int32, (8, 128), 1)
    ninf = jnp.full((8, 128), NEG, jnp.float32)
    pinf = jnp.full((8, 128), POS, jnp.float32)

    def group(ct, g):
        leaves = []
        for j in range(8):
            r0 = g * 32 + j * 4
            vs = [x_ref[0, ct, r0 + i] for i in range(4)]
            leaves.append(_sort4(vs, _c2d))
        tsubs = list(leaves)
        while len(tsubs) > 1:
            tsubs = [_merge4(tsubs[i], tsubs[i + 1], jnp.maximum, _c2d)
                     for i in range(0, len(tsubs), 2)]
        bsubs = [lv[::-1] for lv in leaves]
        while len(bsubs) > 1:
            bsubs = [_merge4(bsubs[i], bsubs[i + 1], jnp.minimum, _c2a)
                     for i in range(0, len(bsubs), 2)]
        return tsubs[0], bsubs[0]

    top = [None, None]
    bot = [None, None]
    for g in range(8):
        for ct in range(2):
            tg, bg = group(ct, g)
            top[ct] = (tg if top[ct] is None
                       else _merge4(top[ct], tg, jnp.maximum, _c2d))
            bot[ct] = (bg if bot[ct] is None
                       else _merge4(bot[ct], bg, jnp.minimum, _c2a))

    for ct in range(2):
        t, bt = top[ct], bot[ct]
        for k in range(4):
            m = jnp.max(t[0], axis=1, keepdims=True)
            mb = jnp.broadcast_to(m, (8, 128))
            li = jnp.min(jnp.where(t[0] == mb, iota, 128), axis=1,
                         keepdims=True)
            sel = iota == jnp.broadcast_to(li, (8, 128))
            o_ref[0, ct, 0, k] = mb
            t = [jnp.where(sel, t[1], t[0]), jnp.where(sel, t[2], t[1]),
                 jnp.where(sel, t[3], t[2]), jnp.where(sel, ninf, t[3])]
        for k in range(4):
            m = jnp.min(bt[0], axis=1, keepdims=True)
            mb = jnp.broadcast_to(m, (8, 128))
            li = jnp.min(jnp.where(bt[0] == mb, iota, 128), axis=1,
                         keepdims=True)
            sel = iota == jnp.broadcast_to(li, (8, 128))
            o_ref[0, ct, 1, k] = mb
            bt = [jnp.where(sel, bt[1], bt[0]), jnp.where(sel, bt[2], bt[1]),
                  jnp.where(sel, bt[3], bt[2]), jnp.where(sel, pinf, bt[3])]


def _milpool_tc(x5):
    return pl.pallas_call(
        _tc_body,
        grid=(BTC,),
        in_specs=[pl.BlockSpec((1, 2, NT, 8, 128),
                               lambda i: (i + BSC, 0, 0, 0, 0))],
        out_specs=pl.BlockSpec((1, 2, 2, 4, 8, 128),
                               lambda i: (i, 0, 0, 0, 0, 0)),
        out_shape=jax.ShapeDtypeStruct((BTC, 2, 2, 4, 8, 128), jnp.float32),
    )(x5)


@jax.jit
def kernel(inputs):
    x5 = inputs.reshape(B, NT, 128, 2, 8).transpose(0, 3, 1, 4, 2)
    o = _milpool_sc(x5)
    otc = _milpool_tc(x5)
    o6 = o.reshape(BSC, 2, 8, 2, 16)[:, :, :, :, :K]
    out_sc = jnp.transpose(o6, (0, 3, 4, 1, 2)).reshape(BSC, 2 * K, C)
    o7 = otc[:, :, :, :, :, 0]
    out_tc = jnp.transpose(o7, (0, 2, 3, 1, 4)).reshape(BTC, 2 * K, C)
    return jnp.concatenate([out_sc, out_tc], axis=0)
